# baseline (speedup 1.0000x reference)
import functools

import jax
import jax.numpy as jnp
from jax import lax
from jax.experimental import pallas as pl
from jax.experimental.pallas import tpu as pltpu
from jax.experimental.pallas import tpu_sc as plsc

_NW = 32
_L = 16
_ROUND_C = float(1.5 * 2.0 ** 23)
_TPAD = 80
_ROWS_W = 96
_CHR = 48
_NCH = 2
_W = 384
_VPC = _CHR * _W // _L


def _sc_body(x_hbm, s_hbm, m_hbm, cat_hbm, out_hbm,
             cat_v, xb0, xb1, sb0, sb1, mb0, mb1,
             isem0, isem1, osem0, osem1):
    wid = lax.axis_index("s") * 2 + lax.axis_index("c")
    bidx = wid // 4
    r0 = (wid % 4) * _ROWS_W

    hcat = pltpu.async_copy(cat_hbm, cat_v, osem1)

    xbufs, sbufs, mbufs = (xb0, xb1), (sb0, sb1), (mb0, mb1)
    isems, osems = (isem0, isem1), (osem0, osem1)

    def sl(ref, c):
        return ref.at[bidx, pl.ds(r0 + c * _CHR, _CHR), :]

    def fire_in(c):
        b = c % 2
        return (pltpu.async_copy(sl(x_hbm, c), xbufs[b], isems[b]),
                pltpu.async_copy(sl(s_hbm, c), sbufs[b], isems[b]),
                pltpu.async_copy(sl(m_hbm, c), mbufs[b], isems[b]))

    hin = fire_in(0)
    hcat.wait()
    av = cat_v[3 * _TPAD:3 * _TPAD + _L]
    bv = cat_v[3 * _TPAD + _L:3 * _TPAD + 2 * _L]
    hout = []
    for c in range(_NCH):
        b = c % 2
        nxt = fire_in(c + 1) if c + 1 < _NCH else None
        for h in hin:
            h.wait()
        xb, sb, mb = xbufs[b], sbufs[b], mbufs[b]

        @plsc.parallel_loop(0, _VPC, 1, unroll=8)
        def vec(i):
            r = i // 24
            coff = pl.multiple_of(i * _L - r * _W, _L)
            xv = xb[r, pl.ds(coff, _L)]
            sv = sb[r, pl.ds(coff, _L)]
            mv = mb[r, pl.ds(coff, _L)]
            sa = jnp.abs(sv)
            bits = lax.bitcast_convert_type(sa, jnp.int32)
            u = bits.astype(jnp.float32) * av + bv
            w = jnp.clip(u.astype(jnp.int32), 0, 62)
            t0 = plsc.load_gather(cat_v, [w])
            t1 = plsc.load_gather(cat_v, [w + _TPAD])
            t2 = plsc.load_gather(cat_v, [w + 2 * _TPAD])
            d0 = sa - t0
            d1 = sa - t1
            d2 = sa - t2
            qs = jnp.where(d0 + d1 > 0.0, t1, t0)
            qs = jnp.where(d1 + d2 > 0.0, t2, qs)
            v = (xv - mv) / qs
            rr = (v + _ROUND_C) - _ROUND_C
            xb[r, pl.ds(coff, _L)] = rr * qs + mv

        hout.append(pltpu.async_copy(xb, sl(out_hbm, c), osems[b]))
        hin = nxt
    for h in hout:
        h.wait()


@jax.jit
def kernel(inputs, scale, mean, scale_table):
    B, H, W = inputs.shape

    pad = jnp.full((_TPAD - 64,), 1e30, jnp.float32)
    t0_tab = jnp.concatenate([scale_table, pad])
    t1_tab = jnp.concatenate([scale_table[1:], pad, pad[:1]])
    t2_tab = jnp.concatenate([scale_table[2:], pad, pad[:2]])

    t0 = scale_table[0]
    t63 = scale_table[63]
    dlog2 = (jnp.log2(t63) - jnp.log2(t0)) * jnp.float32(1.0 / 63.0)
    a = jnp.float32(1.0 / 8388608.0) / dlog2
    b = -(jnp.float32(126.9569643) + jnp.log2(t0)) / dlog2 - jnp.float32(1.0)
    cat = jnp.concatenate([t0_tab, t1_tab, t2_tab,
                           jnp.full((_L,), a, jnp.float32),
                           jnp.full((_L,), b, jnp.float32)])

    mesh = plsc.VectorSubcoreMesh(core_axis_name="c", subcore_axis_name="s")
    fn = functools.partial(
        pl.kernel,
        mesh=mesh,
        out_type=jax.ShapeDtypeStruct((B, H, W), jnp.float32),
        compiler_params=pltpu.CompilerParams(needs_layout_passes=False),
        scratch_types=[
            pltpu.VMEM((3 * _TPAD + 2 * _L,), jnp.float32),
            pltpu.VMEM((_CHR, _W), jnp.float32),
            pltpu.VMEM((_CHR, _W), jnp.float32),
            pltpu.VMEM((_CHR, _W), jnp.float32),
            pltpu.VMEM((_CHR, _W), jnp.float32),
            pltpu.VMEM((_CHR, _W), jnp.float32),
            pltpu.VMEM((_CHR, _W), jnp.float32),
            pltpu.SemaphoreType.DMA,
            pltpu.SemaphoreType.DMA,
            pltpu.SemaphoreType.DMA,
            pltpu.SemaphoreType.DMA,
        ],
    )(_sc_body)
    return fn(inputs, scale, mean, cat)


# speedup vs baseline: 1.0469x; 1.0469x over previous
import functools

import jax
import jax.numpy as jnp
from jax import lax
from jax.experimental import pallas as pl
from jax.experimental.pallas import tpu as pltpu
from jax.experimental.pallas import tpu_sc as plsc

_NW = 32
_L = 16
_ROUND_C = float(1.5 * 2.0 ** 23)
_TPAD = 80
_ROWS_W = 96
_CHR = 32
_NCH = 3
_W = 384
_VPC = _CHR * _W // _L


def _sc_body(x_hbm, s_hbm, m_hbm, cat_hbm, out_hbm,
             t0_v, t1_v, t2_v, cst_v, xb0, xb1, sb0, sb1, mb0, mb1, ob0, ob1,
             isem0, isem1, osem0, osem1):
    wid = lax.axis_index("s") * 2 + lax.axis_index("c")
    bidx = wid // 4
    r0 = (wid % 4) * _ROWS_W

    hcat = (pltpu.async_copy(cat_hbm.at[pl.ds(0, _TPAD)], t0_v, osem0),
            pltpu.async_copy(cat_hbm.at[pl.ds(_TPAD, _TPAD)], t1_v, osem0),
            pltpu.async_copy(cat_hbm.at[pl.ds(2 * _TPAD, _TPAD)], t2_v, osem0),
            pltpu.async_copy(cat_hbm.at[pl.ds(3 * _TPAD, 2 * _L)], cst_v, osem0))

    xbufs, sbufs, mbufs, obufs = (xb0, xb1), (sb0, sb1), (mb0, mb1), (ob0, ob1)
    isems, osems = (isem0, isem1), (osem0, osem1)

    def sl(ref, c):
        return ref.at[bidx, pl.ds(r0 + c * _CHR, _CHR), :]

    def fire_in(c):
        b = c % 2
        return (pltpu.async_copy(sl(x_hbm, c), xbufs[b], isems[b]),
                pltpu.async_copy(sl(s_hbm, c), sbufs[b], isems[b]),
                pltpu.async_copy(sl(m_hbm, c), mbufs[b], isems[b]))

    hin = fire_in(0)
    for h in hcat:
        h.wait()
    av = cst_v[0:_L]
    bv = cst_v[_L:2 * _L]
    hout = [None, None]
    for c in range(_NCH):
        b = c % 2
        nxt = fire_in(c + 1) if c + 1 < _NCH else None
        for h in hin:
            h.wait()
        if hout[b] is not None:
            hout[b].wait()
        xb, sb, mb, ob = xbufs[b], sbufs[b], mbufs[b], obufs[b]

        @plsc.parallel_loop(0, _VPC, 1, unroll=8)
        def vec(i):
            r = i // 24
            coff = pl.multiple_of(i * _L - r * _W, _L)
            xv = xb[r, pl.ds(coff, _L)]
            sv = sb[r, pl.ds(coff, _L)]
            mv = mb[r, pl.ds(coff, _L)]
            sa = jnp.abs(sv)
            bits = lax.bitcast_convert_type(sa, jnp.int32)
            u = bits.astype(jnp.float32) * av + bv
            w = jnp.clip(u.astype(jnp.int32), 0, 62)
            t0 = plsc.load_gather(t0_v, [w])
            t1 = plsc.load_gather(t1_v, [w])
            t2 = plsc.load_gather(t2_v, [w])
            d0 = sa - t0
            d1 = sa - t1
            d2 = sa - t2
            qs = jnp.where(d0 + d1 > 0.0, t1, t0)
            qs = jnp.where(d1 + d2 > 0.0, t2, qs)
            v = (xv - mv) / qs
            rr = (v + _ROUND_C) - _ROUND_C
            ob[r, pl.ds(coff, _L)] = rr * qs + mv

        hout[b] = pltpu.async_copy(ob, sl(out_hbm, c), osems[b])
        hin = nxt
    for h in hout:
        if h is not None:
            h.wait()


@jax.jit
def kernel(inputs, scale, mean, scale_table):
    B, H, W = inputs.shape

    pad = jnp.full((_TPAD - 64,), 1e30, jnp.float32)
    t0_tab = jnp.concatenate([scale_table, pad])
    t1_tab = jnp.concatenate([scale_table[1:], pad, pad[:1]])
    t2_tab = jnp.concatenate([scale_table[2:], pad, pad[:2]])

    t0 = scale_table[0]
    t63 = scale_table[63]
    dlog2 = (jnp.log2(t63) - jnp.log2(t0)) * jnp.float32(1.0 / 63.0)
    a = jnp.float32(1.0 / 8388608.0) / dlog2
    b = -(jnp.float32(126.9569643) + jnp.log2(t0)) / dlog2 - jnp.float32(1.0)
    cat = jnp.concatenate([t0_tab, t1_tab, t2_tab,
                           jnp.full((_L,), a, jnp.float32),
                           jnp.full((_L,), b, jnp.float32)])

    mesh = plsc.VectorSubcoreMesh(core_axis_name="c", subcore_axis_name="s")
    fn = functools.partial(
        pl.kernel,
        mesh=mesh,
        out_type=jax.ShapeDtypeStruct((B, H, W), jnp.float32),
        compiler_params=pltpu.CompilerParams(needs_layout_passes=False),
        scratch_types=[
            pltpu.VMEM((_TPAD,), jnp.float32),
            pltpu.VMEM((_TPAD,), jnp.float32),
            pltpu.VMEM((_TPAD,), jnp.float32),
            pltpu.VMEM((2 * _L,), jnp.float32),
            pltpu.VMEM((_CHR, _W), jnp.float32),
            pltpu.VMEM((_CHR, _W), jnp.float32),
            pltpu.VMEM((_CHR, _W), jnp.float32),
            pltpu.VMEM((_CHR, _W), jnp.float32),
            pltpu.VMEM((_CHR, _W), jnp.float32),
            pltpu.VMEM((_CHR, _W), jnp.float32),
            pltpu.VMEM((_CHR, _W), jnp.float32),
            pltpu.VMEM((_CHR, _W), jnp.float32),
            pltpu.SemaphoreType.DMA,
            pltpu.SemaphoreType.DMA,
            pltpu.SemaphoreType.DMA,
            pltpu.SemaphoreType.DMA,
        ],
    )(_sc_body)
    return fn(inputs, scale, mean, cat)
